# baseline (speedup 1.0000x reference)
import jax
import jax.numpy as jnp
from jax import lax
from jax.experimental import pallas as pl
from jax.experimental.pallas import tpu as pltpu
from jax.experimental.pallas import tpu_sc as plsc

_N = 8192
_NW = 32
_QPW = _N // _NW
_WSCALE = 1e4
_WPAD = 4e4
_BIG = 3.4e38


def _first_geq(w_r, thresh):
    def step(_, lohi):
        lo, hi = lohi
        mid = lax.div(lo + hi, jnp.int32(2))
        v = w_r[pl.ds(mid, 16)][0]
        lt = v < thresh
        return (jnp.where(lt, mid + 1, lo), jnp.where(lt, hi, mid))

    lo, _ = lax.fori_loop(0, 14, step, (jnp.int32(0), jnp.int32(_N)))
    return lo


def _tree_min(vs):
    while len(vs) > 1:
        vs = [jnp.minimum(vs[i], vs[i + 1]) for i in range(0, len(vs) - 1, 2)] \
             + ([vs[-1]] if len(vs) % 2 else [])
    return vs[0]


def _sc_min_body(ax_h, ay_h, az_h, aw_h, bx_h, by_h, bz_h, bw_h,
                 oq_h, od_h,
                 ax, ay, az, aw, bx, by, bz, bw, an2, dm, oq_v):
    wid = lax.axis_index("s") * 2 + lax.axis_index("c")
    qbase = wid * _QPW

    pltpu.sync_copy(ax_h, ax)
    pltpu.sync_copy(ay_h, ay)
    pltpu.sync_copy(az_h, az)
    pltpu.sync_copy(aw_h, aw.at[pl.ds(0, _N)])
    pltpu.sync_copy(bx_h, bx)
    pltpu.sync_copy(by_h, by)
    pltpu.sync_copy(bz_h, bz)
    pltpu.sync_copy(bw_h, bw.at[pl.ds(0, _N)])
    pad = jnp.full((16,), _WPAD, dtype=jnp.float32)
    aw[pl.ds(_N, 16)] = pad
    bw[pl.ds(_N, 16)] = pad

    s1 = _first_geq(aw, 0.5 * _WSCALE)
    s2 = _first_geq(aw, 1.5 * _WSCALE)
    s3 = _first_geq(aw, 2.5 * _WSCALE)

    big = jnp.full((16,), _BIG, dtype=jnp.float32)

    def prep(c, carry):
        off = c * 16
        vx = ax[pl.ds(off, 16)]
        vy = ay[pl.ds(off, 16)]
        vz = az[pl.ds(off, 16)]
        an2[pl.ds(off, 16)] = vx * vx + vy * vy + vz * vz
        dm[pl.ds(off, 16)] = big
        return carry

    lax.fori_loop(0, _N // 16, prep, 0)

    QU = 4

    def qgroup(t, carry):
        qoff = t * 16
        qxv = bx[pl.ds(qbase + qoff, 16)]
        qyv = by[pl.ds(qbase + qoff, 16)]
        qzv = bz[pl.ds(qbase + qoff, 16)]
        qwv = bw[pl.ds(qbase + qoff, 16)]

        wf, wl = qwv[0], qwv[15]
        lo = jnp.where(wf > 2.5 * _WSCALE, s3,
                       jnp.where(wf > 1.5 * _WSCALE, s2,
                                 jnp.where(wf > 0.5 * _WSCALE, s1,
                                           jnp.int32(0))))
        hi = jnp.where(wl > 2.5 * _WSCALE, jnp.int32(_N),
                       jnp.where(wl > 1.5 * _WSCALE, s3,
                                 jnp.where(wl > 0.5 * _WSCALE, s2, s1)))
        c0 = lax.shift_right_logical(lo, 4)
        c1 = lax.shift_right_logical(hi + 15, 4)
        ce = jnp.maximum(c1 - 1, c0)
        ci0 = c0 + 1
        ci1 = jnp.maximum(c1 - 1, ci0)
        same = wf == wl

        for j0 in range(0, 16, QU):
            qs = [(qxv[j0 + u], qyv[j0 + u], qzv[j0 + u], qwv[j0 + u])
                  for u in range(QU)]
            qd = [(-2.0 * sx, -2.0 * sy, -2.0 * sz,
                   sx * sx + sy * sy + sz * sz)
                  for sx, sy, sz, _ in qs]

            def chunk(c, accs, masked):
                off = c * 16
                vx = ax[pl.ds(off, 16)]
                vy = ay[pl.ds(off, 16)]
                vz = az[pl.ds(off, 16)]
                vn2 = an2[pl.ds(off, 16)]
                vw = aw[pl.ds(off, 16)] if masked else None
                out, d2s = [], []
                for u in range(QU):
                    mx, my, mz, q2 = qd[u]
                    if masked:
                        dw = qs[u][3] - vw
                        d2 = (mx * vx + my * vy + mz * vz
                              + (vn2 + dw * dw) + q2)
                    else:
                        d2 = mx * vx + my * vy + mz * vz + (vn2 + q2)
                    d2s.append(d2)
                    out.append(jnp.minimum(accs[u], d2))
                dmv = dm[pl.ds(off, 16)]
                dm[pl.ds(off, 16)] = jnp.minimum(dmv, _tree_min(d2s))
                return tuple(out)

            def sweep(use_w):
                def run(_):
                    acc0 = jnp.full((16,), _BIG, dtype=jnp.float32)
                    accs = chunk(c0, (acc0,) * QU, True)
                    accs = chunk(ce, accs, True)
                    accs = plsc.parallel_loop(ci0, ci1, unroll=4,
                                              carry=accs)(
                        lambda c, a: chunk(c, a, use_w))
                    for u in range(QU):
                        oq_v[pl.ds((qoff + j0 + u) * 16, 16)] = accs[u]
                    return jnp.int32(0)
                return run

            lax.cond(same, sweep(False), sweep(True), jnp.int32(0))
        return carry

    lax.fori_loop(0, _QPW // 16, qgroup, 0)

    pltpu.sync_copy(oq_v, oq_h.at[pl.ds(qbase * 16, _QPW * 16)])
    pltpu.sync_copy(dm, od_h.at[pl.ds(wid * _N, _N)])


_sc_min = pl.kernel(
    _sc_min_body,
    out_type=(jax.ShapeDtypeStruct((_N * 16,), jnp.float32),
              jax.ShapeDtypeStruct((_NW * _N,), jnp.float32)),
    mesh=plsc.VectorSubcoreMesh(core_axis_name="c", subcore_axis_name="s"),
    scratch_types=[pltpu.VMEM((_N,), jnp.float32)] * 3
                  + [pltpu.VMEM((_N + 16,), jnp.float32)]
                  + [pltpu.VMEM((_N,), jnp.float32)] * 3
                  + [pltpu.VMEM((_N + 16,), jnp.float32)]
                  + [pltpu.VMEM((_N,), jnp.float32)] * 2
                  + [pltpu.VMEM((_QPW * 16,), jnp.float32)],
)


def _tc_reduce_body(q_ref, d_ref, o_ref):
    sq = jnp.sum(jnp.sqrt(jnp.maximum(jnp.min(q_ref[...], axis=1), 0.0)))
    sd = jnp.sum(jnp.sqrt(jnp.maximum(jnp.min(d_ref[...], axis=0), 0.0)))
    o_ref[0, 0] = (sq + sd) * (0.5 / _N)


_tc_reduce = pl.pallas_call(
    _tc_reduce_body,
    out_shape=jax.ShapeDtypeStruct((1, 1), jnp.float32),
    out_specs=pl.BlockSpec(memory_space=pltpu.SMEM),
)


def kernel(xyz, xyz_gt, batch_xyz, batch_xyz_gt):
    aw = batch_xyz.astype(jnp.float32) * _WSCALE
    bw = batch_xyz_gt.astype(jnp.float32) * _WSCALE
    min_q, min_d = _sc_min(xyz[:, 0], xyz[:, 1], xyz[:, 2], aw,
                           xyz_gt[:, 0], xyz_gt[:, 1], xyz_gt[:, 2], bw)
    loss = _tc_reduce(min_q.reshape(_N, 16), min_d.reshape(_NW, _N))
    return loss[0, 0]


# speedup vs baseline: 1.1681x; 1.1681x over previous
import jax
import jax.numpy as jnp
from jax import lax
from jax.experimental import pallas as pl
from jax.experimental.pallas import tpu as pltpu
from jax.experimental.pallas import tpu_sc as plsc

_N = 8192
_NW = 32
_QPW = _N // _NW
_WSCALE = 1e4
_WPAD = 4e4
_BIG = 3.4e38


def _first_geq(w_r, thresh):
    def step(_, lohi):
        lo, hi = lohi
        mid = lax.div(lo + hi, jnp.int32(2))
        v = w_r[pl.ds(mid, 16)][0]
        lt = v < thresh
        return (jnp.where(lt, mid + 1, lo), jnp.where(lt, hi, mid))

    lo, _ = lax.fori_loop(0, 14, step, (jnp.int32(0), jnp.int32(_N)))
    return lo


def _tree_min(vs):
    while len(vs) > 1:
        vs = [jnp.minimum(vs[i], vs[i + 1]) for i in range(0, len(vs) - 1, 2)] \
             + ([vs[-1]] if len(vs) % 2 else [])
    return vs[0]


def _sc_min_body(ax_h, ay_h, az_h, aw_h, bx_h, by_h, bz_h, bw_h,
                 oq_h, od_h,
                 ax, ay, az, aw, bx, by, bz, bw, an2, dm, oq_v):
    wid = lax.axis_index("s") * 2 + lax.axis_index("c")
    qbase = wid * _QPW

    pltpu.sync_copy(ax_h, ax)
    pltpu.sync_copy(ay_h, ay)
    pltpu.sync_copy(az_h, az)
    pltpu.sync_copy(aw_h, aw.at[pl.ds(0, _N)])
    pltpu.sync_copy(bx_h, bx)
    pltpu.sync_copy(by_h, by)
    pltpu.sync_copy(bz_h, bz)
    pltpu.sync_copy(bw_h, bw.at[pl.ds(0, _N)])
    pad = jnp.full((16,), _WPAD, dtype=jnp.float32)
    aw[pl.ds(_N, 16)] = pad
    bw[pl.ds(_N, 16)] = pad

    s1 = _first_geq(aw, 0.5 * _WSCALE)
    s2 = _first_geq(aw, 1.5 * _WSCALE)
    s3 = _first_geq(aw, 2.5 * _WSCALE)

    big = jnp.full((16,), _BIG, dtype=jnp.float32)

    def prep(c, carry):
        off = c * 16
        vx = ax[pl.ds(off, 16)]
        vy = ay[pl.ds(off, 16)]
        vz = az[pl.ds(off, 16)]
        an2[pl.ds(off, 16)] = vx * vx + vy * vy + vz * vz
        dm[pl.ds(off, 16)] = big
        return carry

    lax.fori_loop(0, _N // 16, prep, 0)

    QU = 8

    def qgroup(t, carry):
        qoff = t * 16
        qxv = bx[pl.ds(qbase + qoff, 16)]
        qyv = by[pl.ds(qbase + qoff, 16)]
        qzv = bz[pl.ds(qbase + qoff, 16)]
        qwv = bw[pl.ds(qbase + qoff, 16)]

        wf, wl = qwv[0], qwv[15]
        lo = jnp.where(wf > 2.5 * _WSCALE, s3,
                       jnp.where(wf > 1.5 * _WSCALE, s2,
                                 jnp.where(wf > 0.5 * _WSCALE, s1,
                                           jnp.int32(0))))
        hi = jnp.where(wl > 2.5 * _WSCALE, jnp.int32(_N),
                       jnp.where(wl > 1.5 * _WSCALE, s3,
                                 jnp.where(wl > 0.5 * _WSCALE, s2, s1)))
        c0 = lax.shift_right_logical(lo, 4)
        c1 = lax.shift_right_logical(hi + 15, 4)
        ce = jnp.maximum(c1 - 1, c0)
        ci0 = c0 + 1
        ci1 = jnp.maximum(c1 - 1, ci0)
        same = wf == wl

        for j0 in range(0, 16, QU):
            qs = [(qxv[j0 + u], qyv[j0 + u], qzv[j0 + u], qwv[j0 + u])
                  for u in range(QU)]
            qd = [(-2.0 * sx, -2.0 * sy, -2.0 * sz,
                   sx * sx + sy * sy + sz * sz)
                  for sx, sy, sz, _ in qs]

            def chunk(c, accs, masked):
                off = c * 16
                vx = ax[pl.ds(off, 16)]
                vy = ay[pl.ds(off, 16)]
                vz = az[pl.ds(off, 16)]
                vn2 = an2[pl.ds(off, 16)]
                vw = aw[pl.ds(off, 16)] if masked else None
                out, d2s = [], []
                for u in range(QU):
                    mx, my, mz, q2 = qd[u]
                    if masked:
                        dw = qs[u][3] - vw
                        d2 = (mx * vx + my * vy + mz * vz
                              + (vn2 + dw * dw) + q2)
                    else:
                        d2 = mx * vx + my * vy + mz * vz + (vn2 + q2)
                    d2s.append(d2)
                    out.append(jnp.minimum(accs[u], d2))
                dmv = dm[pl.ds(off, 16)]
                dm[pl.ds(off, 16)] = jnp.minimum(dmv, _tree_min(d2s))
                return tuple(out)

            def sweep(use_w):
                def run(_):
                    acc0 = jnp.full((16,), _BIG, dtype=jnp.float32)
                    accs = chunk(c0, (acc0,) * QU, True)
                    accs = chunk(ce, accs, True)
                    accs = plsc.parallel_loop(ci0, ci1, unroll=8,
                                              carry=accs)(
                        lambda c, a: chunk(c, a, use_w))
                    for u in range(QU):
                        oq_v[pl.ds((qoff + j0 + u) * 16, 16)] = accs[u]
                    return jnp.int32(0)
                return run

            lax.cond(same, sweep(False), sweep(True), jnp.int32(0))
        return carry

    lax.fori_loop(0, _QPW // 16, qgroup, 0)

    pltpu.sync_copy(oq_v, oq_h.at[pl.ds(qbase * 16, _QPW * 16)])
    pltpu.sync_copy(dm, od_h.at[pl.ds(wid * _N, _N)])


_sc_min = pl.kernel(
    _sc_min_body,
    out_type=(jax.ShapeDtypeStruct((_N * 16,), jnp.float32),
              jax.ShapeDtypeStruct((_NW * _N,), jnp.float32)),
    mesh=plsc.VectorSubcoreMesh(core_axis_name="c", subcore_axis_name="s"),
    scratch_types=[pltpu.VMEM((_N,), jnp.float32)] * 3
                  + [pltpu.VMEM((_N + 16,), jnp.float32)]
                  + [pltpu.VMEM((_N,), jnp.float32)] * 3
                  + [pltpu.VMEM((_N + 16,), jnp.float32)]
                  + [pltpu.VMEM((_N,), jnp.float32)] * 2
                  + [pltpu.VMEM((_QPW * 16,), jnp.float32)],
)


def _tc_reduce_body(q_ref, d_ref, o_ref):
    sq = jnp.sum(jnp.sqrt(jnp.maximum(jnp.min(q_ref[...], axis=1), 0.0)))
    sd = jnp.sum(jnp.sqrt(jnp.maximum(jnp.min(d_ref[...], axis=0), 0.0)))
    o_ref[0, 0] = (sq + sd) * (0.5 / _N)


_tc_reduce = pl.pallas_call(
    _tc_reduce_body,
    out_shape=jax.ShapeDtypeStruct((1, 1), jnp.float32),
    out_specs=pl.BlockSpec(memory_space=pltpu.SMEM),
)


def kernel(xyz, xyz_gt, batch_xyz, batch_xyz_gt):
    aw = batch_xyz.astype(jnp.float32) * _WSCALE
    bw = batch_xyz_gt.astype(jnp.float32) * _WSCALE
    min_q, min_d = _sc_min(xyz[:, 0], xyz[:, 1], xyz[:, 2], aw,
                           xyz_gt[:, 0], xyz_gt[:, 1], xyz_gt[:, 2], bw)
    loss = _tc_reduce(min_q.reshape(_N, 16), min_d.reshape(_NW, _N))
    return loss[0, 0]


# speedup vs baseline: 1.3534x; 1.1586x over previous
import jax
import jax.numpy as jnp
from jax import lax
from jax.experimental import pallas as pl
from jax.experimental.pallas import tpu as pltpu
from jax.experimental.pallas import tpu_sc as plsc

_N = 8192
_NW = 32
_QPW = _N // _NW
_WSCALE = 1e4
_WPAD = 4e4
_BIG = 3.4e38


def _first_geq(w_r, thresh):
    def step(_, lohi):
        lo, hi = lohi
        mid = lax.div(lo + hi, jnp.int32(2))
        v = w_r[pl.ds(mid, 16)][0]
        lt = v < thresh
        return (jnp.where(lt, mid + 1, lo), jnp.where(lt, hi, mid))

    lo, _ = lax.fori_loop(0, 14, step, (jnp.int32(0), jnp.int32(_N)))
    return lo


def _tree_min(vs):
    while len(vs) > 1:
        vs = [jnp.minimum(vs[i], vs[i + 1]) for i in range(0, len(vs) - 1, 2)] \
             + ([vs[-1]] if len(vs) % 2 else [])
    return vs[0]


def _sc_min_body(ax_h, ay_h, az_h, aw_h, bx_h, by_h, bz_h, bw_h,
                 oq_h, od_h,
                 ax, ay, az, aw, bx, by, bz, bw, an2, dm, oq_v):
    wid = lax.axis_index("s") * 2 + lax.axis_index("c")
    qbase = wid * _QPW

    pltpu.sync_copy(ax_h, ax.at[pl.ds(0, _N)])
    pltpu.sync_copy(ay_h, ay.at[pl.ds(0, _N)])
    pltpu.sync_copy(az_h, az.at[pl.ds(0, _N)])
    pltpu.sync_copy(aw_h, aw.at[pl.ds(0, _N)])
    pltpu.sync_copy(bx_h, bx)
    pltpu.sync_copy(by_h, by)
    pltpu.sync_copy(bz_h, bz)
    pltpu.sync_copy(bw_h, bw.at[pl.ds(0, _N)])
    pad = jnp.full((16,), _WPAD, dtype=jnp.float32)
    aw[pl.ds(_N, 16)] = pad
    bw[pl.ds(_N, 16)] = pad

    s1 = _first_geq(aw, 0.5 * _WSCALE)
    s2 = _first_geq(aw, 1.5 * _WSCALE)
    s3 = _first_geq(aw, 2.5 * _WSCALE)

    big = jnp.full((16,), _BIG, dtype=jnp.float32)

    def prep(c, carry):
        off = c * 16
        vx = ax[pl.ds(off, 16)]
        vy = ay[pl.ds(off, 16)]
        vz = az[pl.ds(off, 16)]
        an2[pl.ds(off, 16)] = vx * vx + vy * vy + vz * vz
        dm[pl.ds(off, 16)] = big
        return carry

    lax.fori_loop(0, _N // 16, prep, 0)

    QU = 8

    def qgroup(t, carry):
        qoff = t * 16
        qxv = bx[pl.ds(qbase + qoff, 16)]
        qyv = by[pl.ds(qbase + qoff, 16)]
        qzv = bz[pl.ds(qbase + qoff, 16)]
        qwv = bw[pl.ds(qbase + qoff, 16)]

        wf, wl = qwv[0], qwv[15]
        lo = jnp.where(wf > 2.5 * _WSCALE, s3,
                       jnp.where(wf > 1.5 * _WSCALE, s2,
                                 jnp.where(wf > 0.5 * _WSCALE, s1,
                                           jnp.int32(0))))
        hi = jnp.where(wl > 2.5 * _WSCALE, jnp.int32(_N),
                       jnp.where(wl > 1.5 * _WSCALE, s3,
                                 jnp.where(wl > 0.5 * _WSCALE, s2, s1)))
        c0 = lax.shift_right_logical(lo, 4)
        c1 = lax.shift_right_logical(hi + 15, 4)
        ce = jnp.maximum(c1 - 1, c0)
        ci0 = c0 + 1
        ci1 = jnp.maximum(c1 - 1, ci0)
        same = wf == wl

        for j0 in range(0, 16, QU):
            qs = [(qxv[j0 + u], qyv[j0 + u], qzv[j0 + u], qwv[j0 + u])
                  for u in range(QU)]
            qd = [(-2.0 * sx, -2.0 * sy, -2.0 * sz,
                   sx * sx + sy * sy + sz * sz)
                  for sx, sy, sz, _ in qs]

            def load_vecs(off, masked):
                return (ax[pl.ds(off, 16)], ay[pl.ds(off, 16)],
                        az[pl.ds(off, 16)], an2[pl.ds(off, 16)],
                        aw[pl.ds(off, 16)] if masked else None)

            def compute(off, vecs, accs, masked):
                vx, vy, vz, vn2, vw = vecs
                out, d2s = [], []
                for u in range(QU):
                    mx, my, mz, q2 = qd[u]
                    if masked:
                        dw = qs[u][3] - vw
                        d2 = (mx * vx + my * vy + mz * vz
                              + (vn2 + dw * dw) + q2)
                    else:
                        d2 = mx * vx + my * vy + mz * vz + (vn2 + q2)
                    d2s.append(d2)
                    out.append(jnp.minimum(accs[u], d2))
                dmv = dm[pl.ds(off, 16)]
                dm[pl.ds(off, 16)] = jnp.minimum(dmv, _tree_min(d2s))
                return tuple(out)

            def chunk(c, accs, masked):
                return compute(c * 16, load_vecs(c * 16, masked), accs,
                               masked)

            def sweep(use_w):
                def run(_):
                    acc0 = jnp.full((16,), _BIG, dtype=jnp.float32)
                    accs = chunk(c0, (acc0,) * QU, True)
                    accs = chunk(ce, accs, True)

                    def body(c, st):
                        accs, vecs = st
                        nvecs = load_vecs(c * 16 + 16, use_w)
                        return (compute(c * 16, vecs, accs, use_w), nvecs)

                    pre = load_vecs(ci0 * 16, use_w)
                    accs, _ = plsc.parallel_loop(ci0, ci1, unroll=4,
                                                 carry=(accs, pre))(body)
                    for u in range(QU):
                        oq_v[pl.ds((qoff + j0 + u) * 16, 16)] = accs[u]
                    return jnp.int32(0)
                return run

            lax.cond(same, sweep(False), sweep(True), jnp.int32(0))
        return carry

    lax.fori_loop(0, _QPW // 16, qgroup, 0)

    pltpu.sync_copy(oq_v, oq_h.at[pl.ds(qbase * 16, _QPW * 16)])
    pltpu.sync_copy(dm, od_h.at[pl.ds(wid * _N, _N)])


_sc_min = pl.kernel(
    _sc_min_body,
    out_type=(jax.ShapeDtypeStruct((_N * 16,), jnp.float32),
              jax.ShapeDtypeStruct((_NW * _N,), jnp.float32)),
    mesh=plsc.VectorSubcoreMesh(core_axis_name="c", subcore_axis_name="s"),
    scratch_types=[pltpu.VMEM((_N + 16,), jnp.float32)] * 4
                  + [pltpu.VMEM((_N,), jnp.float32)] * 3
                  + [pltpu.VMEM((_N + 16,), jnp.float32)]
                  + [pltpu.VMEM((_N + 16,), jnp.float32)]
                  + [pltpu.VMEM((_N,), jnp.float32)]
                  + [pltpu.VMEM((_QPW * 16,), jnp.float32)],
)


def _tc_reduce_body(q_ref, d_ref, o_ref):
    sq = jnp.sum(jnp.sqrt(jnp.maximum(jnp.min(q_ref[...], axis=1), 0.0)))
    sd = jnp.sum(jnp.sqrt(jnp.maximum(jnp.min(d_ref[...], axis=0), 0.0)))
    o_ref[0, 0] = (sq + sd) * (0.5 / _N)


_tc_reduce = pl.pallas_call(
    _tc_reduce_body,
    out_shape=jax.ShapeDtypeStruct((1, 1), jnp.float32),
    out_specs=pl.BlockSpec(memory_space=pltpu.SMEM),
)


def kernel(xyz, xyz_gt, batch_xyz, batch_xyz_gt):
    aw = batch_xyz.astype(jnp.float32) * _WSCALE
    bw = batch_xyz_gt.astype(jnp.float32) * _WSCALE
    min_q, min_d = _sc_min(xyz[:, 0], xyz[:, 1], xyz[:, 2], aw,
                           xyz_gt[:, 0], xyz_gt[:, 1], xyz_gt[:, 2], bw)
    loss = _tc_reduce(min_q.reshape(_N, 16), min_d.reshape(_NW, _N))
    return loss[0, 0]
